# Initial kernel scaffold; baseline (speedup 1.0000x reference)
import jax
import jax.numpy as jnp
from jax.experimental import pallas as pl


def kernel(m, z, x, w, b, ln_m_g, ln_m_b, ln_z_g, ln_z_b):
    raise NotImplementedError("write your pallas kernel here")


# speedup vs baseline: 1.4540x; 1.4540x over previous
import jax
import jax.numpy as jnp
import numpy as np
from jax.experimental import pallas as pl
from jax.experimental.pallas import tpu as pltpu

C_M, C_Z = 256, 128
MIN_BIN, MAX_BIN, NO_BINS = 3.25, 20.75, 15
INF = 100000000.0
LN_EPS = 1e-5

N = 768
BI = 8
NI = N // BI
M = BI * N


def _make_bounds():
    bins = np.linspace(MIN_BIN, MAX_BIN, NO_BINS).astype(np.float32)
    sq = bins * bins
    lower = np.concatenate([sq, np.float32([2e9])])
    upper = np.concatenate([sq[1:], np.float32([INF, 3e9])])
    return lower, upper


_LOWER, _UPPER = _make_bounds()
_HI = jax.lax.Precision.HIGHEST


def _fused_kernel(m_ref, z_ref, xt_ref, s_ref, xjf_ref, wt_ref, b_ref,
                  gm_ref, bm_ref, gz_ref, bz_ref, lo_ref, up_ref,
                  m_out_ref, z_out_ref):
    mt = m_ref[...]
    mu = jnp.mean(mt, axis=-1, keepdims=True)
    mc = mt - mu
    var = jnp.mean(mc * mc, axis=-1, keepdims=True)
    m_out_ref[...] = mc * jax.lax.rsqrt(var + LN_EPS) * gm_ref[...] + bm_ref[...]

    zt = z_ref[...].reshape(M, C_Z)
    zmu = jnp.mean(zt, axis=-1, keepdims=True)
    zc = zt - zmu
    zvar = jnp.mean(zc * zc, axis=-1, keepdims=True)
    zn = zc * jax.lax.rsqrt(zvar + LN_EPS) * gz_ref[...] + bz_ref[...]

    xif = jnp.dot(xt_ref[0], s_ref[...],
                  preferred_element_type=jnp.float32, precision=_HI)
    df = xif - xjf_ref[...]
    dist2 = jnp.sum(df * df, axis=0, keepdims=True)

    ohf = ((dist2 > lo_ref[...]) & (dist2 < up_ref[...])).astype(jnp.float32)
    dp = jax.lax.dot_general(ohf, wt_ref[...], (((0,), (0,)), ((), ())),
                             preferred_element_type=jnp.float32, precision=_HI)

    z_out_ref[...] = (zn + dp + b_ref[...]).reshape(BI, N, C_Z)


def kernel(m, z, x, w, b, ln_m_g, ln_m_b, ln_z_g, ln_z_b):
    m2 = m.reshape(N, C_M)
    z2 = z.reshape(N, N, C_Z)
    x2 = x.reshape(N, 3)

    xt = jnp.zeros((8, N), jnp.float32).at[:3, :].set(x2.T)
    xtb = xt.reshape(8, NI, BI).transpose(1, 0, 2)
    xjf = jnp.tile(xt, (1, BI))
    seg = np.repeat(np.arange(BI), N)
    s = jnp.asarray((seg[None, :] == np.arange(BI)[:, None]).astype(np.float32))
    wt = jnp.zeros((16, C_Z), jnp.float32).at[:NO_BINS, :].set(w.T)

    m_up, z_up = pl.pallas_call(
        _fused_kernel,
        grid=(NI,),
        in_specs=[
            pl.BlockSpec((BI, C_M), lambda i: (i, 0)),
            pl.BlockSpec((BI, N, C_Z), lambda i: (i, 0, 0)),
            pl.BlockSpec((1, 8, BI), lambda i: (i, 0, 0)),
            pl.BlockSpec((BI, M), lambda i: (0, 0)),
            pl.BlockSpec((8, M), lambda i: (0, 0)),
            pl.BlockSpec((16, C_Z), lambda i: (0, 0)),
            pl.BlockSpec((1, C_Z), lambda i: (0, 0)),
            pl.BlockSpec((1, C_M), lambda i: (0, 0)),
            pl.BlockSpec((1, C_M), lambda i: (0, 0)),
            pl.BlockSpec((1, C_Z), lambda i: (0, 0)),
            pl.BlockSpec((1, C_Z), lambda i: (0, 0)),
            pl.BlockSpec((16, 1), lambda i: (0, 0)),
            pl.BlockSpec((16, 1), lambda i: (0, 0)),
        ],
        out_specs=[
            pl.BlockSpec((BI, C_M), lambda i: (i, 0)),
            pl.BlockSpec((BI, N, C_Z), lambda i: (i, 0, 0)),
        ],
        out_shape=[
            jax.ShapeDtypeStruct((N, C_M), jnp.float32),
            jax.ShapeDtypeStruct((N, N, C_Z), jnp.float32),
        ],
        compiler_params=pltpu.CompilerParams(
            dimension_semantics=("parallel",),
        ),
        name="recycling_embedder_fused",
    )(m2, z2, xtb, s, xjf, wt, b.reshape(1, C_Z),
      ln_m_g.reshape(1, C_M), ln_m_b.reshape(1, C_M),
      ln_z_g.reshape(1, C_Z), ln_z_b.reshape(1, C_Z),
      jnp.asarray(_LOWER).reshape(16, 1), jnp.asarray(_UPPER).reshape(16, 1))

    return m_up.reshape(m.shape), z_up.reshape(z.shape)


# speedup vs baseline: 1.8611x; 1.2800x over previous
import jax
import jax.numpy as jnp
import numpy as np
from jax.experimental import pallas as pl
from jax.experimental.pallas import tpu as pltpu

C_M, C_Z = 256, 128
MIN_BIN, MAX_BIN, NO_BINS = 3.25, 20.75, 15
INF = 100000000.0
LN_EPS = 1e-5

N = 768
BI = 8
NI = N // BI
M = BI * N


def _make_bounds():
    bins = np.linspace(MIN_BIN, MAX_BIN, NO_BINS).astype(np.float32)
    sq = bins * bins
    lower = np.concatenate([sq, np.float32([-1.0])])
    upper = np.concatenate([sq[1:], np.float32([INF, 1e30])])
    return lower, upper


_LOWER, _UPPER = _make_bounds()
_HI = jax.lax.Precision.HIGHEST


def _fused_kernel(m_ref, z_ref, xt_ref, s_ref, xjf_ref, wt_ref,
                  gm_ref, bm_ref, gz_ref, lo_ref, up_ref,
                  m_out_ref, z_out_ref):
    mt = m_ref[...]
    mu = jnp.mean(mt, axis=-1, keepdims=True)
    mc = mt - mu
    var = jnp.mean(mc * mc, axis=-1, keepdims=True)
    m_out_ref[...] = mc * jax.lax.rsqrt(var + LN_EPS) * gm_ref[...] + bm_ref[...]

    xif = jnp.dot(xt_ref[0], s_ref[...],
                  preferred_element_type=jnp.float32, precision=_HI)
    df = xif - xjf_ref[...]
    dist2 = jnp.sum(df * df, axis=0, keepdims=True)

    ohf = ((dist2 > lo_ref[...]) & (dist2 < up_ref[...])).astype(jnp.bfloat16)
    dp = jax.lax.dot_general(ohf, wt_ref[...], (((0,), (0,)), ((), ())),
                             preferred_element_type=jnp.float32)

    zt = z_ref[...].reshape(M, C_Z)
    zmu = jnp.mean(zt, axis=-1, keepdims=True)
    zms = jnp.mean(zt * zt, axis=-1, keepdims=True)
    a = jax.lax.rsqrt(zms - zmu * zmu + LN_EPS)
    zn = (zt - zmu) * a * gz_ref[...]

    z_out_ref[...] = (zn + dp).reshape(BI, N, C_Z)


def kernel(m, z, x, w, b, ln_m_g, ln_m_b, ln_z_g, ln_z_b):
    m2 = m.reshape(N, C_M)
    z2 = z.reshape(N, N, C_Z)
    x2 = x.reshape(N, 3)

    xt = jnp.zeros((8, N), jnp.float32).at[:3, :].set(x2.T)
    xtb = xt.reshape(8, NI, BI).transpose(1, 0, 2)
    xjf = jnp.tile(xt, (1, BI))
    seg = np.repeat(np.arange(BI), N)
    s = jnp.asarray((seg[None, :] == np.arange(BI)[:, None]).astype(np.float32))
    wt = (jnp.zeros((16, C_Z), jnp.float32)
          .at[:NO_BINS, :].set(w.T)
          .at[NO_BINS, :].set(b + ln_z_b)).astype(jnp.bfloat16)

    m_up, z_up = pl.pallas_call(
        _fused_kernel,
        grid=(NI,),
        in_specs=[
            pl.BlockSpec((BI, C_M), lambda i: (i, 0)),
            pl.BlockSpec((BI, N, C_Z), lambda i: (i, 0, 0)),
            pl.BlockSpec((1, 8, BI), lambda i: (i, 0, 0)),
            pl.BlockSpec((BI, M), lambda i: (0, 0)),
            pl.BlockSpec((8, M), lambda i: (0, 0)),
            pl.BlockSpec((16, C_Z), lambda i: (0, 0)),
            pl.BlockSpec((1, C_M), lambda i: (0, 0)),
            pl.BlockSpec((1, C_M), lambda i: (0, 0)),
            pl.BlockSpec((1, C_Z), lambda i: (0, 0)),
            pl.BlockSpec((16, 1), lambda i: (0, 0)),
            pl.BlockSpec((16, 1), lambda i: (0, 0)),
        ],
        out_specs=[
            pl.BlockSpec((BI, C_M), lambda i: (i, 0)),
            pl.BlockSpec((BI, N, C_Z), lambda i: (i, 0, 0)),
        ],
        out_shape=[
            jax.ShapeDtypeStruct((N, C_M), jnp.float32),
            jax.ShapeDtypeStruct((N, N, C_Z), jnp.float32),
        ],
        compiler_params=pltpu.CompilerParams(
            dimension_semantics=("parallel",),
        ),
        name="recycling_embedder_fused",
    )(m2, z2, xtb, s, xjf, wt,
      ln_m_g.reshape(1, C_M), ln_m_b.reshape(1, C_M),
      ln_z_g.reshape(1, C_Z),
      jnp.asarray(_LOWER).reshape(16, 1), jnp.asarray(_UPPER).reshape(16, 1))

    return m_up.reshape(m.shape), z_up.reshape(z.shape)


# speedup vs baseline: 2.1514x; 1.1559x over previous
import jax
import jax.numpy as jnp
import numpy as np
from jax.experimental import pallas as pl
from jax.experimental.pallas import tpu as pltpu

C_M, C_Z = 256, 128
MIN_BIN, MAX_BIN, NO_BINS = 3.25, 20.75, 15
INF = 100000000.0
LN_EPS = 1e-5

N = 768
BI = 16
NI = N // BI
M = BI * N


def _make_bounds():
    bins = np.linspace(MIN_BIN, MAX_BIN, NO_BINS).astype(np.float32)
    sq = bins * bins
    lower = np.concatenate([sq, np.float32([-1.0])])
    upper = np.concatenate([sq[1:], np.float32([INF, 1e30])])
    return lower, upper


_LOWER, _UPPER = _make_bounds()
_HI = jax.lax.Precision.HIGHEST


def _fused_kernel(m_ref, z_ref, xt_ref, s_ref, xjf_ref, wt_ref,
                  gm_ref, bm_ref, gz_ref, lo_ref, up_ref,
                  m_out_ref, z_out_ref):
    mt = m_ref[...]
    mu = jnp.mean(mt, axis=-1, keepdims=True)
    mc = mt - mu
    var = jnp.mean(mc * mc, axis=-1, keepdims=True)
    m_out_ref[...] = mc * jax.lax.rsqrt(var + LN_EPS) * gm_ref[...] + bm_ref[...]

    xif = jnp.dot(xt_ref[0], s_ref[...],
                  preferred_element_type=jnp.float32, precision=_HI)
    df = xif - xjf_ref[...]
    dist2 = jnp.sum(df * df, axis=0, keepdims=True)

    ohf = ((dist2 > lo_ref[...]) & (dist2 < up_ref[...])).astype(jnp.bfloat16)
    dp = jax.lax.dot_general(ohf, wt_ref[...], (((0,), (0,)), ((), ())),
                             preferred_element_type=jnp.float32)

    zt = z_ref[...].reshape(M, C_Z)
    s1 = jnp.sum(zt, axis=-1, keepdims=True)
    s2 = jnp.sum(zt * zt, axis=-1, keepdims=True)
    u = s2 * jnp.float32(C_Z) - s1 * s1 + jnp.float32(C_Z * C_Z * LN_EPS)
    r = jax.lax.rsqrt(u)
    zn = (zt * jnp.float32(C_Z) - s1) * r * gz_ref[...]

    z_out_ref[...] = (zn + dp).reshape(BI, N, C_Z)


def kernel(m, z, x, w, b, ln_m_g, ln_m_b, ln_z_g, ln_z_b):
    m2 = m.reshape(N, C_M)
    z2 = z.reshape(N, N, C_Z)
    x2 = x.reshape(N, 3)

    xt = jnp.zeros((8, N), jnp.float32).at[:3, :].set(x2.T)
    xtb = xt.reshape(8, NI, BI).transpose(1, 0, 2)
    xjf = jnp.tile(xt, (1, BI))
    seg = np.repeat(np.arange(BI), N)
    s = jnp.asarray((seg[None, :] == np.arange(BI)[:, None]).astype(np.float32))
    wt = (jnp.zeros((16, C_Z), jnp.float32)
          .at[:NO_BINS, :].set(w.T)
          .at[NO_BINS, :].set(b + ln_z_b)).astype(jnp.bfloat16)

    m_up, z_up = pl.pallas_call(
        _fused_kernel,
        grid=(NI,),
        in_specs=[
            pl.BlockSpec((BI, C_M), lambda i: (i, 0)),
            pl.BlockSpec((BI, N, C_Z), lambda i: (i, 0, 0)),
            pl.BlockSpec((1, 8, BI), lambda i: (i, 0, 0)),
            pl.BlockSpec((BI, M), lambda i: (0, 0)),
            pl.BlockSpec((8, M), lambda i: (0, 0)),
            pl.BlockSpec((16, C_Z), lambda i: (0, 0)),
            pl.BlockSpec((1, C_M), lambda i: (0, 0)),
            pl.BlockSpec((1, C_M), lambda i: (0, 0)),
            pl.BlockSpec((1, C_Z), lambda i: (0, 0)),
            pl.BlockSpec((16, 1), lambda i: (0, 0)),
            pl.BlockSpec((16, 1), lambda i: (0, 0)),
        ],
        out_specs=[
            pl.BlockSpec((BI, C_M), lambda i: (i, 0)),
            pl.BlockSpec((BI, N, C_Z), lambda i: (i, 0, 0)),
        ],
        out_shape=[
            jax.ShapeDtypeStruct((N, C_M), jnp.float32),
            jax.ShapeDtypeStruct((N, N, C_Z), jnp.float32),
        ],
        compiler_params=pltpu.CompilerParams(
            dimension_semantics=("parallel",),
        ),
        name="recycling_embedder_fused",
    )(m2, z2, xtb, s, xjf, wt,
      ln_m_g.reshape(1, C_M), ln_m_b.reshape(1, C_M),
      ln_z_g.reshape(1, C_Z),
      jnp.asarray(_LOWER).reshape(16, 1), jnp.asarray(_UPPER).reshape(16, 1))

    return m_up.reshape(m.shape), z_up.reshape(z.shape)


# speedup vs baseline: 2.3358x; 1.0857x over previous
import jax
import jax.numpy as jnp
import numpy as np
from jax.experimental import pallas as pl
from jax.experimental.pallas import tpu as pltpu

C_M, C_Z = 256, 128
MIN_BIN, MAX_BIN, NO_BINS = 3.25, 20.75, 15
INF = 100000000.0
LN_EPS = 1e-5

N = 768
BI = 16
NI = N // BI
M = BI * N
NC = 8
RC = BI // NC
MC = RC * N


def _make_bounds():
    bins = np.linspace(MIN_BIN, MAX_BIN, NO_BINS).astype(np.float32)
    sq = bins * bins
    lower = np.concatenate([sq, np.float32([-1.0])])
    upper = np.concatenate([sq[1:], np.float32([INF, 1e30])])
    return lower, upper


_LOWER, _UPPER = _make_bounds()
_HI = jax.lax.Precision.HIGHEST


def _fused_kernel(m_ref, z_ref, xt_ref, s_ref, xjf_ref, wt_ref, ones_ref,
                  gm_ref, bm_ref, gz_ref, lo_ref, up_ref,
                  m_out_ref, z_out_ref):
    mt = m_ref[...]
    mu = jnp.mean(mt, axis=-1, keepdims=True)
    mc = mt - mu
    var = jnp.mean(mc * mc, axis=-1, keepdims=True)
    m_out_ref[...] = mc * jax.lax.rsqrt(var + LN_EPS) * gm_ref[...] + bm_ref[...]

    xif = jnp.dot(xt_ref[0], s_ref[...],
                  preferred_element_type=jnp.float32, precision=_HI)
    df = xif - xjf_ref[...]
    dist2 = jnp.sum(df * df, axis=0, keepdims=True)

    ohf = ((dist2 > lo_ref[...]) & (dist2 < up_ref[...])).astype(jnp.bfloat16)

    gz = gz_ref[...]
    ones = ones_ref[...]
    for cc in range(NC):
        mc = slice(cc * MC, (cc + 1) * MC)
        dpc = jax.lax.dot_general(ohf[:, mc], wt_ref[...],
                                  (((0,), (0,)), ((), ())),
                                  preferred_element_type=jnp.float32)
        zt = z_ref[cc * RC:(cc + 1) * RC].reshape(MC, C_Z)
        zb = zt.astype(jnp.bfloat16)
        s1b = jnp.dot(zb, ones, preferred_element_type=jnp.float32)
        s2b = jnp.dot(zb * zb, ones, preferred_element_type=jnp.float32)
        u = s2b * jnp.float32(C_Z) - s1b * s1b + jnp.float32(C_Z * C_Z * LN_EPS)
        r = jax.lax.rsqrt(u)
        zn = (zt * jnp.float32(C_Z) - s1b) * r * gz
        z_out_ref[cc * RC:(cc + 1) * RC] = (zn + dpc).reshape(RC, N, C_Z)


def kernel(m, z, x, w, b, ln_m_g, ln_m_b, ln_z_g, ln_z_b):
    m2 = m.reshape(N, C_M)
    z2 = z.reshape(N, N, C_Z)
    x2 = x.reshape(N, 3)

    xt = jnp.zeros((8, N), jnp.float32).at[:3, :].set(x2.T)
    xtb = xt.reshape(8, NI, BI).transpose(1, 0, 2)
    xjf = jnp.tile(xt, (1, BI))
    seg = np.repeat(np.arange(BI), N)
    s = jnp.asarray((seg[None, :] == np.arange(BI)[:, None]).astype(np.float32))
    wt = (jnp.zeros((16, C_Z), jnp.float32)
          .at[:NO_BINS, :].set(w.T)
          .at[NO_BINS, :].set(b + ln_z_b)).astype(jnp.bfloat16)

    m_up, z_up = pl.pallas_call(
        _fused_kernel,
        grid=(NI,),
        in_specs=[
            pl.BlockSpec((BI, C_M), lambda i: (i, 0)),
            pl.BlockSpec((BI, N, C_Z), lambda i: (i, 0, 0)),
            pl.BlockSpec((1, 8, BI), lambda i: (i, 0, 0)),
            pl.BlockSpec((BI, M), lambda i: (0, 0)),
            pl.BlockSpec((8, M), lambda i: (0, 0)),
            pl.BlockSpec((16, C_Z), lambda i: (0, 0)),
            pl.BlockSpec((C_Z, C_Z), lambda i: (0, 0)),
            pl.BlockSpec((1, C_M), lambda i: (0, 0)),
            pl.BlockSpec((1, C_M), lambda i: (0, 0)),
            pl.BlockSpec((1, C_Z), lambda i: (0, 0)),
            pl.BlockSpec((16, 1), lambda i: (0, 0)),
            pl.BlockSpec((16, 1), lambda i: (0, 0)),
        ],
        out_specs=[
            pl.BlockSpec((BI, C_M), lambda i: (i, 0)),
            pl.BlockSpec((BI, N, C_Z), lambda i: (i, 0, 0)),
        ],
        out_shape=[
            jax.ShapeDtypeStruct((N, C_M), jnp.float32),
            jax.ShapeDtypeStruct((N, N, C_Z), jnp.float32),
        ],
        compiler_params=pltpu.CompilerParams(
            dimension_semantics=("parallel",),
        ),
        name="recycling_embedder_fused",
    )(m2, z2, xtb, s, xjf, wt, jnp.ones((C_Z, C_Z), jnp.bfloat16),
      ln_m_g.reshape(1, C_M), ln_m_b.reshape(1, C_M),
      ln_z_g.reshape(1, C_Z),
      jnp.asarray(_LOWER).reshape(16, 1), jnp.asarray(_UPPER).reshape(16, 1))

    return m_up.reshape(m.shape), z_up.reshape(z.shape)


# speedup vs baseline: 2.4160x; 1.0343x over previous
import jax
import jax.numpy as jnp
import numpy as np
from jax.experimental import pallas as pl
from jax.experimental.pallas import tpu as pltpu

C_M, C_Z = 256, 128
MIN_BIN, MAX_BIN, NO_BINS = 3.25, 20.75, 15
INF = 100000000.0
LN_EPS = 1e-5

N = 768
BI = 24
NI = N // BI
M = BI * N
NC = 8
RC = BI // NC
MC = RC * N


def _make_bounds():
    bins = np.linspace(MIN_BIN, MAX_BIN, NO_BINS).astype(np.float32)
    sq = bins * bins
    lower = np.concatenate([sq, np.float32([-1.0])])
    upper = np.concatenate([sq[1:], np.float32([INF, 1e30])])
    return lower, upper


_LOWER, _UPPER = _make_bounds()
_HI = jax.lax.Precision.HIGHEST


def _fused_kernel(m_ref, z_ref, xt_ref, s_ref, xjf_ref, wt_ref, ones_ref,
                  gm_ref, bm_ref, gz_ref, lo_ref, up_ref,
                  m_out_ref, z_out_ref):
    mt = m_ref[...]
    mu = jnp.mean(mt, axis=-1, keepdims=True)
    mc = mt - mu
    var = jnp.mean(mc * mc, axis=-1, keepdims=True)
    m_out_ref[...] = mc * jax.lax.rsqrt(var + LN_EPS) * gm_ref[...] + bm_ref[...]

    xif = jnp.dot(xt_ref[0], s_ref[...],
                  preferred_element_type=jnp.float32, precision=_HI)
    df = xif - xjf_ref[...]
    dist2 = jnp.sum(df * df, axis=0, keepdims=True)

    ohf = ((dist2 > lo_ref[...]) & (dist2 < up_ref[...])).astype(jnp.bfloat16)

    gz = gz_ref[...]
    ones = ones_ref[...]
    for cc in range(NC):
        mc = slice(cc * MC, (cc + 1) * MC)
        dpc = jax.lax.dot_general(ohf[:, mc], wt_ref[...],
                                  (((0,), (0,)), ((), ())),
                                  preferred_element_type=jnp.float32)
        zt = z_ref[cc * RC:(cc + 1) * RC].reshape(MC, C_Z)
        zb = zt.astype(jnp.bfloat16)
        s1b = jnp.dot(zb, ones, preferred_element_type=jnp.float32)
        s2b = jnp.dot(zb * zb, ones, preferred_element_type=jnp.float32)
        u = s2b * jnp.float32(C_Z) - s1b * s1b + jnp.float32(C_Z * C_Z * LN_EPS)
        r = jax.lax.rsqrt(u)
        zn = (zt * jnp.float32(C_Z) - s1b) * r * gz
        z_out_ref[cc * RC:(cc + 1) * RC] = (zn + dpc).reshape(RC, N, C_Z)


def kernel(m, z, x, w, b, ln_m_g, ln_m_b, ln_z_g, ln_z_b):
    m2 = m.reshape(N, C_M)
    z2 = z.reshape(N, N, C_Z)
    x2 = x.reshape(N, 3)

    xt = jnp.zeros((8, N), jnp.float32).at[:3, :].set(x2.T)
    xtb = xt.reshape(8, NI, BI).transpose(1, 0, 2)
    xjf = jnp.tile(xt, (1, BI))
    seg = np.repeat(np.arange(BI), N)
    s = jnp.asarray((seg[None, :] == np.arange(BI)[:, None]).astype(np.float32))
    wt = (jnp.zeros((16, C_Z), jnp.float32)
          .at[:NO_BINS, :].set(w.T)
          .at[NO_BINS, :].set(b + ln_z_b)).astype(jnp.bfloat16)

    m_up, z_up = pl.pallas_call(
        _fused_kernel,
        grid=(NI,),
        in_specs=[
            pl.BlockSpec((BI, C_M), lambda i: (i, 0)),
            pl.BlockSpec((BI, N, C_Z), lambda i: (i, 0, 0)),
            pl.BlockSpec((1, 8, BI), lambda i: (i, 0, 0)),
            pl.BlockSpec((BI, M), lambda i: (0, 0)),
            pl.BlockSpec((8, M), lambda i: (0, 0)),
            pl.BlockSpec((16, C_Z), lambda i: (0, 0)),
            pl.BlockSpec((C_Z, C_Z), lambda i: (0, 0)),
            pl.BlockSpec((1, C_M), lambda i: (0, 0)),
            pl.BlockSpec((1, C_M), lambda i: (0, 0)),
            pl.BlockSpec((1, C_Z), lambda i: (0, 0)),
            pl.BlockSpec((16, 1), lambda i: (0, 0)),
            pl.BlockSpec((16, 1), lambda i: (0, 0)),
        ],
        out_specs=[
            pl.BlockSpec((BI, C_M), lambda i: (i, 0)),
            pl.BlockSpec((BI, N, C_Z), lambda i: (i, 0, 0)),
        ],
        out_shape=[
            jax.ShapeDtypeStruct((N, C_M), jnp.float32),
            jax.ShapeDtypeStruct((N, N, C_Z), jnp.float32),
        ],
        compiler_params=pltpu.CompilerParams(
            dimension_semantics=("parallel",),
        ),
        name="recycling_embedder_fused",
    )(m2, z2, xtb, s, xjf, wt, jnp.ones((C_Z, C_Z), jnp.bfloat16),
      ln_m_g.reshape(1, C_M), ln_m_b.reshape(1, C_M),
      ln_z_g.reshape(1, C_Z),
      jnp.asarray(_LOWER).reshape(16, 1), jnp.asarray(_UPPER).reshape(16, 1))

    return m_up.reshape(m.shape), z_up.reshape(z.shape)
